# baseline (device time: 23141 ns/iter reference)
import os

import numpy as np
import jax
import jax.numpy as jnp
from jax import lax
from jax.experimental import pallas as pl
from jax.experimental.pallas import tpu as pltpu

N_DEV = 8
B, SQ, D = 2, 128, 512
HQL, DH = 4, 64
DLOC = HQL * DH
BSQ = B * SQ
SCALE = 0.125

_BF16 = jnp.bfloat16
_F32 = jnp.float32


def _rope_tables():
    inv = 1.0 / (10000.0 ** (np.arange(0, DH, 2) / DH))
    pos = np.arange(SQ)[:, None] * inv[None, :]
    cos = np.repeat(np.cos(pos), 2, axis=-1)
    sin = np.repeat(np.sin(pos), 2, axis=-1)
    cos_full = np.tile(cos, (B, HQL)).astype(np.float32)
    sin_full = np.tile(sin, (B, HQL)).astype(np.float32)
    R = np.zeros((DLOC, DLOC), np.float32)
    for c in range(0, DLOC, 2):
        R[c + 1, c] = -1.0
        R[c, c + 1] = 1.0
    return cos_full, sin_full, R


def kernel(x, Wq, Wk, Wv, Wo):
    cos_np, sin_np, rot_np = _rope_tables()
    cos = jnp.asarray(cos_np)
    sin = jnp.asarray(sin_np)
    rot = jnp.asarray(rot_np).astype(_BF16)

    x2 = x.reshape(BSQ, D).astype(_BF16)
    wq = (Wq * (SCALE * float(np.log2(np.e)))).astype(_BF16)
    wk = Wk.astype(_BF16)
    wv = Wv.astype(_BF16)
    wo = Wo.astype(_BF16)

    def body(x_ref, wq_ref, wk_ref, wv_ref, wo_ref, cos_ref, sin_ref,
             rot_ref, out_ref, send_buf, recv_buf, send_sems, recv_sems):
        my = lax.axis_index("i")
        partners = [my ^ (1 << k) for k in range(3)]

        barrier = pltpu.get_barrier_semaphore()
        for nbr in partners:
            pl.semaphore_signal(
                barrier, inc=1,
                device_id=(nbr,), device_id_type=pl.DeviceIdType.MESH,
            )
        pl.semaphore_wait(barrier, 3)

        skip_comm = os.environ.get("SKIP_COMM") == "1"

        def exchange(k, b):
            return pltpu.make_async_remote_copy(
                src_ref=send_buf.at[b],
                dst_ref=recv_buf.at[k, b],
                send_sem=send_sems.at[k, b],
                recv_sem=recv_sems.at[k, b],
                device_id=(partners[k],),
                device_id_type=pl.DeviceIdType.MESH,
            )

        x2b = x_ref[:]
        q2 = jnp.dot(x2b, wq_ref[:], preferred_element_type=_F32)
        k2 = jnp.dot(x2b, wk_ref[:], preferred_element_type=_F32)
        v2 = jnp.dot(x2b, wv_ref[:], preferred_element_type=_F32)
        qr = jnp.dot(q2.astype(_BF16), rot_ref[:], preferred_element_type=_F32)
        kr = jnp.dot(k2.astype(_BF16), rot_ref[:], preferred_element_type=_F32)
        qb = (q2 * cos_ref[:] + qr * sin_ref[:]).astype(_BF16)
        kb = (k2 * cos_ref[:] + kr * sin_ref[:]).astype(_BF16)
        vb = v2.astype(_BF16)

        for b in range(B):
            r0 = b * SQ
            ctxs = []
            for h in range(HQL):
                c0 = h * DH
                qh = qb[r0:r0 + SQ, c0:c0 + DH]
                kh = kb[r0:r0 + SQ, c0:c0 + DH]
                vh = vb[r0:r0 + SQ, c0:c0 + DH]
                s = lax.dot_general(
                    qh, kh, (((1,), (1,)), ((), ())),
                    preferred_element_type=_F32,
                )
                w = jnp.exp2(s)
                r = 1.0 / jnp.sum(w, axis=-1, keepdims=True)
                ctx = jnp.dot(w.astype(_BF16), vh, preferred_element_type=_F32)
                ctxs.append(ctx * r)
            ctx_b = jnp.concatenate(ctxs, axis=1).astype(_BF16)
            part = jnp.dot(ctx_b, wo_ref[:], preferred_element_type=_F32)
            out_ref[b] = part
            send_buf[b] = part.astype(_BF16)
            if not skip_comm:
                exchange(0, b).start()

        if skip_comm:
            return

        for k in range(3):
            for b in range(B):
                exchange(k, b).wait()
                acc = out_ref[b] + recv_buf[k, b].astype(_F32)
                out_ref[b] = acc
                if k < 2:
                    send_buf[b] = acc.astype(_BF16)
                    exchange(k + 1, b).start()

    return pl.pallas_call(
        body,
        out_shape=jax.ShapeDtypeStruct((B, SQ, D), _F32),
        in_specs=[pl.BlockSpec(memory_space=pltpu.VMEM)] * 8,
        out_specs=pl.BlockSpec(memory_space=pltpu.VMEM),
        scratch_shapes=[
            pltpu.VMEM((B, SQ, D), _BF16),
            pltpu.VMEM((3, B, SQ, D), _BF16),
            pltpu.SemaphoreType.DMA((3, B)),
            pltpu.SemaphoreType.DMA((3, B)),
        ],
        compiler_params=pltpu.CompilerParams(collective_id=0),
    )(x2, wq, wk, wv, wo, cos, sin, rot)


# device time: 21779 ns/iter; 1.0625x vs baseline; 1.0625x over previous
import os

import numpy as np
import jax
import jax.numpy as jnp
from jax import lax
from jax.experimental import pallas as pl
from jax.experimental.pallas import tpu as pltpu

N_DEV = 8
B, SQ, D = 2, 128, 512
HQL, DH = 4, 64
DLOC = HQL * DH
BSQ = B * SQ
SCALE = 0.125

_BF16 = jnp.bfloat16
_F32 = jnp.float32


def _rope_tables():
    inv = 1.0 / (10000.0 ** (np.arange(0, DH, 2) / DH))
    pos = np.arange(SQ)[:, None] * inv[None, :]
    cos = np.repeat(np.cos(pos), 2, axis=-1)
    sin = np.repeat(np.sin(pos), 2, axis=-1)
    cos_full = np.tile(cos, (B, HQL)).astype(np.float32)
    sin_full = np.tile(sin, (B, HQL)).astype(np.float32)
    R = np.zeros((DLOC, DLOC), np.float32)
    for c in range(0, DLOC, 2):
        R[c + 1, c] = -1.0
        R[c, c + 1] = 1.0
    return cos_full, sin_full, R


def kernel(x, Wq, Wk, Wv, Wo):
    cos_np, sin_np, rot_np = _rope_tables()
    cos = jnp.asarray(cos_np)
    sin = jnp.asarray(sin_np)
    rot = jnp.asarray(rot_np).astype(_BF16)

    x2 = x.reshape(BSQ, D).astype(_BF16)
    wq = (Wq * (SCALE * float(np.log2(np.e)))).astype(_BF16)
    wk = Wk.astype(_BF16)
    wv = Wv.astype(_BF16)
    wo = Wo.astype(_BF16)

    def body(x_ref, wq_ref, wk_ref, wv_ref, wo_ref, cos_ref, sin_ref,
             rot_ref, out_ref, send_buf, recv_buf, send_sems, recv_sems):
        my = lax.axis_index("i")
        partners = [my ^ m for m in (2, 1, 4)]

        barrier = pltpu.get_barrier_semaphore()
        for nbr in partners:
            pl.semaphore_signal(
                barrier, inc=1,
                device_id=(nbr,), device_id_type=pl.DeviceIdType.MESH,
            )
        pl.semaphore_wait(barrier, 3)

        skip_comm = os.environ.get("SKIP_COMM") == "1"

        CH = SQ // 2

        def exchange(k, c):
            b, off = c // 2, (c % 2) * CH
            return pltpu.make_async_remote_copy(
                src_ref=send_buf.at[b, pl.ds(off, CH)],
                dst_ref=recv_buf.at[k, b, pl.ds(off, CH)],
                send_sem=send_sems.at[k, c],
                recv_sem=recv_sems.at[k, c],
                device_id=(partners[k],),
                device_id_type=pl.DeviceIdType.MESH,
            )

        x2b = x_ref[:]
        q2 = jnp.dot(x2b, wq_ref[:], preferred_element_type=_F32)
        k2 = jnp.dot(x2b, wk_ref[:], preferred_element_type=_F32)
        v2 = jnp.dot(x2b, wv_ref[:], preferred_element_type=_F32)
        qr = jnp.dot(q2.astype(_BF16), rot_ref[:], preferred_element_type=_F32)
        kr = jnp.dot(k2.astype(_BF16), rot_ref[:], preferred_element_type=_F32)
        qb = (q2 * cos_ref[:] + qr * sin_ref[:]).astype(_BF16)
        kb = (k2 * cos_ref[:] + kr * sin_ref[:]).astype(_BF16)
        vb = v2.astype(_BF16)

        for b in range(B):
            r0 = b * SQ
            ctxs = []
            for h in range(HQL):
                c0 = h * DH
                qh = qb[r0:r0 + SQ, c0:c0 + DH]
                kh = kb[r0:r0 + SQ, c0:c0 + DH]
                vh = vb[r0:r0 + SQ, c0:c0 + DH]
                s = lax.dot_general(
                    qh, kh, (((1,), (1,)), ((), ())),
                    preferred_element_type=_F32,
                )
                w = jnp.exp2(s)
                r = 1.0 / jnp.sum(w, axis=-1, keepdims=True)
                ctx = jnp.dot(w.astype(_BF16), vh, preferred_element_type=_F32)
                ctxs.append(ctx * r)
            ctx_b = jnp.concatenate(ctxs, axis=1).astype(_BF16)
            part = jnp.dot(ctx_b, wo_ref[:], preferred_element_type=_F32)
            out_ref[b] = part
            send_buf[b] = part.astype(_BF16)
            if not skip_comm:
                exchange(0, 2 * b).start()
                exchange(0, 2 * b + 1).start()

        if skip_comm:
            return

        for k in range(3):
            for c in range(4):
                b, off = c // 2, (c % 2) * CH
                rows = pl.ds(off, CH)
                exchange(k, c).wait()
                acc = out_ref[b, rows] + recv_buf[k, b, rows].astype(_F32)
                out_ref[b, rows] = acc
                if k < 2:
                    send_buf[b, rows] = acc.astype(_BF16)
                    exchange(k + 1, c).start()

    return pl.pallas_call(
        body,
        out_shape=jax.ShapeDtypeStruct((B, SQ, D), _F32),
        in_specs=[pl.BlockSpec(memory_space=pltpu.VMEM)] * 8,
        out_specs=pl.BlockSpec(memory_space=pltpu.VMEM),
        scratch_shapes=[
            pltpu.VMEM((B, SQ, D), _BF16),
            pltpu.VMEM((3, B, SQ, D), _BF16),
            pltpu.SemaphoreType.DMA((3, 4)),
            pltpu.SemaphoreType.DMA((3, 4)),
        ],
        compiler_params=pltpu.CompilerParams(collective_id=0),
    )(x2, wq, wk, wv, wo, cos, sin, rot)


# device time: 12674 ns/iter; 1.8259x vs baseline; 1.7184x over previous
import os

import numpy as np
import jax
import jax.numpy as jnp
from jax import lax
from jax.experimental import pallas as pl
from jax.experimental.pallas import tpu as pltpu

N_DEV = 8
B, SQ, D = 2, 128, 512
HQL, DH = 4, 64
DLOC = HQL * DH
BSQ = B * SQ
SCALE = 0.125

_BF16 = jnp.bfloat16
_F32 = jnp.float32


def _rope_tables():
    inv = 1.0 / (10000.0 ** (np.arange(0, DH, 2) / DH))
    pos = np.arange(SQ)[:, None] * inv[None, :]
    cos = np.repeat(np.cos(pos), 2, axis=-1)
    sin = np.repeat(np.sin(pos), 2, axis=-1)
    cos_full = np.tile(cos, (B, HQL)).astype(np.float32)
    sin_full = np.tile(sin, (B, HQL)).astype(np.float32)
    R = np.zeros((DLOC, DLOC), np.float32)
    for c in range(0, DLOC, 2):
        R[c + 1, c] = -1.0
        R[c, c + 1] = 1.0
    return cos_full, sin_full, R


def kernel(x, Wq, Wk, Wv, Wo):
    cos_np, sin_np, rot_np = _rope_tables()
    cos = jnp.asarray(cos_np)
    sin = jnp.asarray(sin_np)
    rot = jnp.asarray(rot_np).astype(_BF16)

    x2 = x.reshape(BSQ, D).astype(_BF16)
    wq = (Wq * (SCALE * float(np.log2(np.e)))).astype(_BF16)
    wk = Wk.astype(_BF16)
    wv = Wv.astype(_BF16)
    wo = Wo.astype(_BF16)

    def body(x_ref, wq_ref, wk_ref, wv_ref, wo_ref, cos_ref, sin_ref,
             rot_ref, out_ref, send_buf, recv_buf, send_sems, recv_sems):
        my = lax.axis_index("i")
        partners = [my ^ m for m in (1, 2, 4)]
        MASKS = ((2, 1, 4), (1, 4, 2), (4, 2, 1), (2, 4, 1))

        barrier = pltpu.get_barrier_semaphore()
        for nbr in partners:
            pl.semaphore_signal(
                barrier, inc=1,
                device_id=(nbr,), device_id_type=pl.DeviceIdType.MESH,
            )
        pl.semaphore_wait(barrier, 3)

        skip_comm = os.environ.get("SKIP_COMM") == "1"

        CH = SQ // 2

        def exchange(k, c):
            b, off = c // 2, (c % 2) * CH
            return pltpu.make_async_remote_copy(
                src_ref=send_buf.at[b, pl.ds(off, CH)],
                dst_ref=recv_buf.at[k, b, pl.ds(off, CH)],
                send_sem=send_sems.at[k, c],
                recv_sem=recv_sems.at[k, c],
                device_id=(my ^ MASKS[c][k],),
                device_id_type=pl.DeviceIdType.MESH,
            )

        x2b = x_ref[:]
        q2 = jnp.dot(x2b, wq_ref[:], preferred_element_type=_F32)
        k2 = jnp.dot(x2b, wk_ref[:], preferred_element_type=_F32)
        v2 = jnp.dot(x2b, wv_ref[:], preferred_element_type=_F32)
        qr = jnp.dot(q2.astype(_BF16), rot_ref[:], preferred_element_type=_F32)
        kr = jnp.dot(k2.astype(_BF16), rot_ref[:], preferred_element_type=_F32)
        qb = (q2 * cos_ref[:] + qr * sin_ref[:]).astype(_BF16)
        kb = (k2 * cos_ref[:] + kr * sin_ref[:]).astype(_BF16)
        vb = v2.astype(_BF16)

        for b in range(B):
            r0 = b * SQ
            ctxs = []
            for h in range(HQL):
                c0 = h * DH
                qh = qb[r0:r0 + SQ, c0:c0 + DH]
                kh = kb[r0:r0 + SQ, c0:c0 + DH]
                vh = vb[r0:r0 + SQ, c0:c0 + DH]
                s = lax.dot_general(
                    qh, kh, (((1,), (1,)), ((), ())),
                    preferred_element_type=_F32,
                )
                w = jnp.exp2(s.astype(_BF16))
                r = 1.0 / jnp.sum(w.astype(_F32), axis=-1, keepdims=True)
                ctx = jnp.dot(w, vh, preferred_element_type=_F32)
                ctxs.append(ctx * r)
            ctx_b = jnp.concatenate(ctxs, axis=1).astype(_BF16)
            part = jnp.dot(ctx_b, wo_ref[:], preferred_element_type=_F32)
            out_ref[b] = part
            send_buf[b] = part.astype(_BF16)
            if not skip_comm:
                exchange(0, 2 * b).start()
                exchange(0, 2 * b + 1).start()

        if skip_comm:
            return

        for k in range(3):
            for c in range(4):
                b, off = c // 2, (c % 2) * CH
                rows = pl.ds(off, CH)
                exchange(k, c).wait()
                acc = out_ref[b, rows] + recv_buf[k, b, rows].astype(_F32)
                out_ref[b, rows] = acc
                if k < 2:
                    send_buf[b, rows] = acc.astype(_BF16)
                    exchange(k + 1, c).start()

    return pl.pallas_call(
        body,
        out_shape=jax.ShapeDtypeStruct((B, SQ, D), _F32),
        in_specs=[pl.BlockSpec(memory_space=pltpu.VMEM)] * 8,
        out_specs=pl.BlockSpec(memory_space=pltpu.VMEM),
        scratch_shapes=[
            pltpu.VMEM((B, SQ, D), _BF16),
            pltpu.VMEM((3, B, SQ, D), _BF16),
            pltpu.SemaphoreType.DMA((3, 4)),
            pltpu.SemaphoreType.DMA((3, 4)),
        ],
        compiler_params=pltpu.CompilerParams(collective_id=0),
    )(x2, wq, wk, wv, wo, cos, sin, rot)
